# baseline (speedup 1.0000x reference)
import jax
import jax.numpy as jnp
from jax import lax
from jax.experimental import pallas as pl
from jax.experimental.pallas import tpu as pltpu
from jax.experimental.pallas import tpu_sc as plsc

B = 16384
L = 200
H = 100
GW = 104

NC = 2
NS = 16
NW = NC * NS
ROWS_W = B // NW
CH = 128
NCHUNK = ROWS_W // CH

_CAST_OFFS = (112, 128, 144, 160, 176, 184)


def _body(x_hbm, table_hbm, out_hbm, x_v, out_v, sem):
    wid = lax.axis_index("s") * NC + lax.axis_index("c")
    lane = lax.iota(jnp.int32, 16)

    def chunk(k, carry):
        base = wid * ROWS_W + k * CH
        pltpu.sync_copy(x_hbm.at[pl.ds(base, CH)], x_v)

        def fire(r, c):
            pltpu.async_copy(
                table_hbm.at[x_v.at[r, pl.ds(0, GW)]],
                out_v.at[r, pl.ds(0, GW)],
                sem,
            )
            return c

        lax.fori_loop(0, CH, fire, 0)

        def cast_row(r, c):
            for o in _CAST_OFFS:
                out_v[r, pl.ds(o, 16)] = x_v[r, pl.ds(o, 16)].astype(
                    jnp.float32)
            return c

        lax.fori_loop(0, CH, cast_row, 0)

        drain = pltpu.make_async_copy(
            table_hbm.at[x_v.at[0, pl.ds(0, GW)]],
            out_v.at[0, pl.ds(0, GW)],
            sem,
        )

        def wait(r, c):
            drain.wait()
            return c

        lax.fori_loop(0, CH, wait, 0)

        def fix_row(r, c):
            kept = out_v[r, pl.ds(96, 16)]
            cast = x_v[r, pl.ds(96, 16)].astype(jnp.float32)
            out_v[r, pl.ds(96, 16)] = jnp.where(lane < 4, kept, cast)
            return c

        lax.fori_loop(0, CH, fix_row, 0)
        pltpu.sync_copy(out_v, out_hbm.at[pl.ds(base, CH)])
        return carry

    lax.fori_loop(0, NCHUNK, chunk, 0)


def kernel(x, table):
    table1 = table.reshape(-1)
    mesh = plsc.VectorSubcoreMesh(core_axis_name="c", subcore_axis_name="s")
    run = pl.kernel(
        _body,
        out_type=jax.ShapeDtypeStruct((B, L), jnp.float32),
        mesh=mesh,
        compiler_params=pltpu.CompilerParams(use_tc_tiling_on_sc=False),
        scratch_types=[
            pltpu.VMEM((CH, L), jnp.int32),
            pltpu.VMEM((CH, L), jnp.float32),
            pltpu.SemaphoreType.DMA,
        ],
    )
    return run(x, table1)


# speedup vs baseline: 1.5613x; 1.5613x over previous
import jax
import jax.numpy as jnp
from jax import lax
from jax.experimental import pallas as pl
from jax.experimental.pallas import tpu as pltpu
from jax.experimental.pallas import tpu_sc as plsc

B = 16384
L = 200
H = 100
N = B * H
V = 1000000

NC = 2
NS = 16
NW = NC * NS
PW = N // NW
CH = 12800
NCH = PW // CH
GL = 128
NG = CH // GL

CS = 65536
HOP = 8192
LAST_OFF = V - CS


def _body(idx_hbm, table_hbm, emb_hbm, idx_v, emb_v, tab_s, sem):
    sid = lax.axis_index("s")
    wid = sid * NC + lax.axis_index("c")
    base = wid * PW

    stage_off = jnp.where(sid == 15, LAST_OFF, sid * CS)
    for h in range(0, CS, HOP):
        off = stage_off + h
        pltpu.sync_copy(table_hbm.at[pl.ds(off, HOP)], emb_v.at[pl.ds(0, HOP)])
        pltpu.sync_copy(emb_v.at[pl.ds(0, HOP)], tab_s.at[pl.ds(off, HOP)])
    plsc.subcore_barrier()

    def chunk(k, carry):
        cbase = base + k * CH
        pltpu.sync_copy(idx_hbm.at[pl.ds(cbase, CH)], idx_v)

        def fire(j, c):
            pltpu.async_copy(
                tab_s.at[idx_v.at[pl.ds(j * GL, GL)]],
                emb_v.at[pl.ds(j * GL, GL)],
                sem,
            )
            return c

        drain = pltpu.make_async_copy(
            tab_s.at[idx_v.at[pl.ds(0, GL)]], emb_v.at[pl.ds(0, GL)], sem
        )

        def wait(j, c):
            drain.wait()
            return c

        lax.fori_loop(0, NG, fire, 0)
        lax.fori_loop(0, NG, wait, 0)
        pltpu.sync_copy(emb_v, emb_hbm.at[pl.ds(cbase, CH)])
        return carry

    lax.fori_loop(0, NCH, chunk, 0)


def kernel(x, table):
    xt = x.T
    idx = xt[:H].reshape(-1)
    table1 = table.reshape(-1)
    mesh = plsc.VectorSubcoreMesh(core_axis_name="c", subcore_axis_name="s")
    run = pl.kernel(
        _body,
        out_type=jax.ShapeDtypeStruct((N,), jnp.float32),
        mesh=mesh,
        scratch_types=[
            pltpu.VMEM((CH,), jnp.int32),
            pltpu.VMEM((CH,), jnp.float32),
            pltpu.VMEM_SHARED((V,), jnp.float32),
            pltpu.SemaphoreType.DMA,
        ],
    )
    emb = run(idx, table1).reshape(H, B)
    outt = jnp.concatenate([emb, xt[H:].astype(jnp.float32)], axis=0)
    return outt.T


# speedup vs baseline: 1.5841x; 1.0146x over previous
import jax
import jax.numpy as jnp
from jax import lax
from jax.experimental import pallas as pl
from jax.experimental.pallas import tpu as pltpu
from jax.experimental.pallas import tpu_sc as plsc

B = 16384
L = 200
H = 100
N = B * H
V = 1000000

NC = 2
NS = 16
NW = NC * NS
PW = N // NW
CH = 25600
NCH = PW // CH
GL = 128
NG = CH // GL

CS = 65536
HOP = 8192
LAST_OFF = V - CS


def _body(idx_hbm, table_hbm, emb_hbm, idx_v, emb_v, tab_s, sem):
    sid = lax.axis_index("s")
    wid = sid * NC + lax.axis_index("c")
    base = wid * PW

    stage_off = jnp.where(sid == 15, LAST_OFF, sid * CS)
    for h in range(0, CS, HOP):
        off = stage_off + h
        pltpu.sync_copy(table_hbm.at[pl.ds(off, HOP)], emb_v.at[pl.ds(0, HOP)])
        pltpu.sync_copy(emb_v.at[pl.ds(0, HOP)], tab_s.at[pl.ds(off, HOP)])
    plsc.subcore_barrier()

    def chunk(k, carry):
        cbase = base + k * CH
        pltpu.sync_copy(idx_hbm.at[pl.ds(cbase, CH)], idx_v)

        def fire(j, c):
            pltpu.async_copy(
                tab_s.at[idx_v.at[pl.ds(j * GL, GL)]],
                emb_v.at[pl.ds(j * GL, GL)],
                sem,
            )
            return c

        drain = pltpu.make_async_copy(
            tab_s.at[idx_v.at[pl.ds(0, GL)]], emb_v.at[pl.ds(0, GL)], sem
        )

        def wait(j, c):
            drain.wait()
            return c

        lax.fori_loop(0, NG, fire, 0)
        lax.fori_loop(0, NG, wait, 0)
        pltpu.sync_copy(emb_v, emb_hbm.at[pl.ds(cbase, CH)])
        return carry

    lax.fori_loop(0, NCH, chunk, 0)


def kernel(x, table):
    xt = x.T
    idx = xt[:H].reshape(-1)
    table1 = table.reshape(-1)
    mesh = plsc.VectorSubcoreMesh(core_axis_name="c", subcore_axis_name="s")
    run = pl.kernel(
        _body,
        out_type=jax.ShapeDtypeStruct((N,), jnp.float32),
        mesh=mesh,
        scratch_types=[
            pltpu.VMEM((CH,), jnp.int32),
            pltpu.VMEM((CH,), jnp.float32),
            pltpu.VMEM_SHARED((V,), jnp.float32),
            pltpu.SemaphoreType.DMA,
        ],
    )
    emb = run(idx, table1).reshape(H, B)
    outt = jnp.concatenate([emb, xt[H:].astype(jnp.float32)], axis=0)
    return outt.T


# speedup vs baseline: 1.6300x; 1.0289x over previous
import jax
import jax.numpy as jnp
from jax import lax
from jax.experimental import pallas as pl
from jax.experimental.pallas import tpu as pltpu
from jax.experimental.pallas import tpu_sc as plsc

B = 16384
L = 200
H = 100
N = B * H
V = 1000000

NC = 2
NS = 16
NW = NC * NS
PW = N // NW
CH = 12800
NCH = PW // CH
GL = 128
NG = CH // GL

CS = 65536
HOP = 8192
LAST_OFF = V - CS


def _body(idx_hbm, table_hbm, emb_hbm, idx0, idx1, emb0, emb1, tab_s,
          sem_g, sem_i, sem_e):
    sid = lax.axis_index("s")
    wid = sid * NC + lax.axis_index("c")
    base = wid * PW

    stage_off = jnp.where(sid == 15, LAST_OFF, sid * CS)
    for h in range(0, CS, HOP):
        off = stage_off + h
        pltpu.sync_copy(table_hbm.at[pl.ds(off, HOP)], emb0.at[pl.ds(0, HOP)])
        pltpu.sync_copy(emb0.at[pl.ds(0, HOP)], tab_s.at[pl.ds(off, HOP)])
    plsc.subcore_barrier()

    idxb = (idx0, idx1)
    embb = (emb0, emb1)
    pltpu.async_copy(idx_hbm.at[pl.ds(base, CH)], idx0, sem_i)

    for k in range(NCH):
        b = k % 2
        pltpu.make_async_copy(
            idx_hbm.at[pl.ds(0, CH)], idxb[b], sem_i
        ).wait()
        if k + 1 < NCH:
            pltpu.async_copy(
                idx_hbm.at[pl.ds(base + (k + 1) * CH, CH)], idxb[1 - b], sem_i
            )
        if k >= 2:
            pltpu.make_async_copy(
                embb[b], emb_hbm.at[pl.ds(0, CH)], sem_e
            ).wait()

        def fire(j, c, b=b):
            pltpu.async_copy(
                tab_s.at[idxb[b].at[pl.ds(j * GL, GL)]],
                embb[b].at[pl.ds(j * GL, GL)],
                sem_g,
            )
            return c

        drain = pltpu.make_async_copy(
            tab_s.at[idxb[b].at[pl.ds(0, GL)]],
            embb[b].at[pl.ds(0, GL)],
            sem_g,
        )

        def wait(j, c):
            drain.wait()
            return c

        lax.fori_loop(0, NG, fire, 0)
        lax.fori_loop(0, NG, wait, 0)
        pltpu.async_copy(embb[b], emb_hbm.at[pl.ds(base + k * CH, CH)], sem_e)

    for b in range(2):
        pltpu.make_async_copy(embb[b], emb_hbm.at[pl.ds(0, CH)], sem_e).wait()


def kernel(x, table):
    xt = x.T
    idx = xt[:H].reshape(-1)
    table1 = table.reshape(-1)
    mesh = plsc.VectorSubcoreMesh(core_axis_name="c", subcore_axis_name="s")
    run = pl.kernel(
        _body,
        out_type=jax.ShapeDtypeStruct((N,), jnp.float32),
        mesh=mesh,
        scratch_types=[
            pltpu.VMEM((CH,), jnp.int32),
            pltpu.VMEM((CH,), jnp.int32),
            pltpu.VMEM((CH,), jnp.float32),
            pltpu.VMEM((CH,), jnp.float32),
            pltpu.VMEM_SHARED((V,), jnp.float32),
            pltpu.SemaphoreType.DMA,
            pltpu.SemaphoreType.DMA,
            pltpu.SemaphoreType.DMA,
        ],
    )
    emb = run(idx, table1).reshape(H, B)
    outt = jnp.concatenate([emb, xt[H:].astype(jnp.float32)], axis=0)
    return outt.T
